# baseline (speedup 1.0000x reference)
import functools

import jax
import jax.numpy as jnp
from jax import lax
from jax.experimental import pallas as pl
from jax.experimental.pallas import tpu as pltpu
from jax.experimental.pallas import tpu_sc as plsc

TREE_NODE_NUM = 2000000
EMBED_DIM = 64
BATCH = 16384
HIST = 50
TOTAL = BATCH * HIST

_info = plsc.get_sparse_core_info()
_NC = _info.num_cores
_NS = _info.num_subcores
_NW = _NC * _NS

CHUNK = 128
K = 4
GROUP = K * CHUNK
NB = 2
PER_W = TOTAL // _NW
NCHUNK = PER_W // CHUNK
NGROUP = PER_W // GROUP
NITER = NGROUP // NB

_mesh = plsc.VectorSubcoreMesh(core_axis_name="c", subcore_axis_name="s")


@functools.partial(
    pl.kernel,
    mesh=_mesh,
    out_type=jax.ShapeDtypeStruct((TOTAL, EMBED_DIM), jnp.float32),
    scratch_types=[
        pltpu.VMEM((NCHUNK, CHUNK), jnp.int32),
        pltpu.VMEM((NB, GROUP, EMBED_DIM), jnp.float32),
        pltpu.SemaphoreType.DMA,
        pltpu.SemaphoreType.DMA,
        pltpu.SemaphoreType.DMA,
        pltpu.SemaphoreType.DMA,
    ],
    compiler_params=pltpu.CompilerParams(use_tc_tiling_on_sc=False),
)
def _sc_gather(idx_hbm, table_hbm, out_hbm, idx_v, rows_v, g0, g1, s0, s1):
    c = lax.axis_index("c")
    s = lax.axis_index("s")
    wid = s * _NC + c
    gsem = (g0, g1)
    ssem = (s0, s1)
    pltpu.sync_copy(idx_hbm.at[wid], idx_v)
    base = wid * PER_W

    def fire_group(gidx, b):
        for k in range(K):
            pltpu.async_copy(
                table_hbm.at[idx_v.at[gidx * K + k]],
                rows_v.at[b, pl.ds(k * CHUNK, CHUNK)],
                gsem[b],
            )

    def drain_gathers(b):
        pltpu.make_async_copy(
            table_hbm.at[pl.ds(0, GROUP)], rows_v.at[b], gsem[b]
        ).wait()

    def drain_store(b):
        pltpu.make_async_copy(
            rows_v.at[b], out_hbm.at[pl.ds(0, GROUP)], ssem[b]
        ).wait()

    for b in range(NB):
        fire_group(b, b)

    def body(i, carry):
        for b in range(NB):
            g = i * NB + b
            drain_gathers(b)
            pltpu.async_copy(
                rows_v.at[b],
                out_hbm.at[pl.ds(base + g * GROUP, GROUP)],
                ssem[b],
            )
            gn = g + NB

            @pl.when(gn < NGROUP)
            def _refill():
                drain_store(b)
                fire_group(gn, b)

        return carry

    lax.fori_loop(0, NITER, body, 0)
    for b in range(NB):
        drain_store(b)


FBT = 3200
FGRID = TREE_NODE_NUM // FBT


def _fmt_body(t_ref, out_ref):
    x = t_ref[...]
    eye = jnp.eye(EMBED_DIM, dtype=jnp.float32)
    xt = jax.lax.dot_general(
        x, eye, (((0,), (0,)), ((), ())),
        preferred_element_type=jnp.float32,
    )
    xt3 = xt.reshape(FBT // 2, 2, EMBED_DIM)
    out_ref[...] = jnp.concatenate([xt3[:, 0, :], xt3[:, 1, :]], axis=1)


_tc_format = pl.pallas_call(
    _fmt_body,
    grid=(FGRID,),
    in_specs=[pl.BlockSpec((EMBED_DIM, FBT), lambda i: (0, i))],
    out_specs=pl.BlockSpec((FBT // 2, 2 * EMBED_DIM), lambda i: (i, 0)),
    out_shape=jax.ShapeDtypeStruct(
        (TREE_NODE_NUM // 2, 2 * EMBED_DIM), jnp.float32
    ),
)


def kernel(itemIDs, emb_table):
    idx = itemIDs.reshape(_NW, NCHUNK, CHUNK).astype(jnp.int32)
    pairs = _tc_format(emb_table.T)
    table_rm = pairs.reshape(TREE_NODE_NUM, EMBED_DIM)
    out = _sc_gather(idx, table_rm)
    return out.reshape(BATCH, HIST, EMBED_DIM)


# speedup vs baseline: 1.1219x; 1.1219x over previous
import functools

import jax
import jax.numpy as jnp
from jax import lax
from jax.experimental import pallas as pl
from jax.experimental.pallas import tpu as pltpu
from jax.experimental.pallas import tpu_sc as plsc

TREE_NODE_NUM = 2000000
EMBED_DIM = 64
BATCH = 16384
HIST = 50
TOTAL = BATCH * HIST

_info = plsc.get_sparse_core_info()
_NC = _info.num_cores
_NS = _info.num_subcores
_NW = _NC * _NS

CHUNK = 128
K = 4
GROUP = K * CHUNK
NB = 2
PER_W = TOTAL // _NW
NCHUNK = PER_W // CHUNK
NGROUP = PER_W // GROUP
NITER = NGROUP // NB

_mesh = plsc.VectorSubcoreMesh(core_axis_name="c", subcore_axis_name="s")


@functools.partial(
    pl.kernel,
    mesh=_mesh,
    out_type=jax.ShapeDtypeStruct((TOTAL, EMBED_DIM), jnp.float32),
    scratch_types=[
        pltpu.VMEM((NCHUNK, CHUNK), jnp.int32),
        pltpu.VMEM((NB, GROUP, EMBED_DIM), jnp.float32),
        pltpu.SemaphoreType.DMA,
        pltpu.SemaphoreType.DMA,
        pltpu.SemaphoreType.DMA,
        pltpu.SemaphoreType.DMA,
    ],
    compiler_params=pltpu.CompilerParams(use_tc_tiling_on_sc=False),
)
def _sc_gather(idx_hbm, table_hbm, out_hbm, idx_v, rows_v, g0, g1, s0, s1):
    c = lax.axis_index("c")
    s = lax.axis_index("s")
    wid = s * _NC + c
    gsem = (g0, g1)
    ssem = (s0, s1)
    pltpu.sync_copy(idx_hbm.at[wid], idx_v)
    base = wid * PER_W

    def fire_group(gidx, b):
        for k in range(K):
            pltpu.async_copy(
                table_hbm.at[idx_v.at[gidx * K + k]],
                rows_v.at[b, pl.ds(k * CHUNK, CHUNK)],
                gsem[b],
            )

    def drain_gathers(b):
        pltpu.make_async_copy(
            table_hbm.at[pl.ds(0, GROUP)], rows_v.at[b], gsem[b]
        ).wait()

    def drain_store(b):
        pltpu.make_async_copy(
            rows_v.at[b], out_hbm.at[pl.ds(0, GROUP)], ssem[b]
        ).wait()

    for b in range(NB):
        fire_group(b, b)

    def body(i, carry):
        for b in range(NB):
            g = i * NB + b
            drain_gathers(b)
            pltpu.async_copy(
                rows_v.at[b],
                out_hbm.at[pl.ds(base + g * GROUP, GROUP)],
                ssem[b],
            )
            gn = g + NB

            @pl.when(gn < NGROUP)
            def _refill():
                drain_store(b)
                fire_group(gn, b)

        return carry

    lax.fori_loop(0, NITER, body, 0)
    for b in range(NB):
        drain_store(b)


FBT = 16000
FGRID = TREE_NODE_NUM // FBT


def _fmt_body(t_ref, out_ref):
    x = t_ref[...]
    xt = jnp.swapaxes(x, 0, 1)
    xt3 = xt.reshape(FBT // 2, 2, EMBED_DIM)
    out_ref[...] = jnp.concatenate([xt3[:, 0, :], xt3[:, 1, :]], axis=1)


_tc_format = pl.pallas_call(
    _fmt_body,
    grid=(FGRID,),
    in_specs=[pl.BlockSpec((EMBED_DIM, FBT), lambda i: (0, i))],
    out_specs=pl.BlockSpec((FBT // 2, 2 * EMBED_DIM), lambda i: (i, 0)),
    out_shape=jax.ShapeDtypeStruct(
        (TREE_NODE_NUM // 2, 2 * EMBED_DIM), jnp.float32
    ),
)


def kernel(itemIDs, emb_table):
    idx = itemIDs.reshape(_NW, NCHUNK, CHUNK).astype(jnp.int32)
    pairs = _tc_format(emb_table.T)
    table_rm = pairs.reshape(TREE_NODE_NUM, EMBED_DIM)
    out = _sc_gather(idx, table_rm)
    return out.reshape(BATCH, HIST, EMBED_DIM)


# speedup vs baseline: 1.3855x; 1.2349x over previous
import functools

import jax
import jax.numpy as jnp
from jax import lax
from jax.experimental import pallas as pl
from jax.experimental.pallas import tpu as pltpu
from jax.experimental.pallas import tpu_sc as plsc

TREE_NODE_NUM = 2000000
EMBED_DIM = 64
BATCH = 16384
HIST = 50
TOTAL = BATCH * HIST

_info = plsc.get_sparse_core_info()
_NC = _info.num_cores
_NS = _info.num_subcores
_NW = _NC * _NS

CHUNK = 128
K = 4
GROUP = K * CHUNK
NB = 2
PER_W = TOTAL // _NW
NCHUNK = PER_W // CHUNK
NGROUP = PER_W // GROUP
NITER = NGROUP // NB

_mesh = plsc.VectorSubcoreMesh(core_axis_name="c", subcore_axis_name="s")


@functools.partial(
    pl.kernel,
    mesh=_mesh,
    out_type=jax.ShapeDtypeStruct((TOTAL, EMBED_DIM), jnp.float32),
    scratch_types=[
        pltpu.VMEM((NCHUNK, CHUNK), jnp.int32),
        pltpu.VMEM((NB, GROUP, EMBED_DIM), jnp.float32),
        pltpu.SemaphoreType.DMA,
        pltpu.SemaphoreType.DMA,
        pltpu.SemaphoreType.DMA,
        pltpu.SemaphoreType.DMA,
    ],
    compiler_params=pltpu.CompilerParams(use_tc_tiling_on_sc=False),
)
def _sc_gather(idx_hbm, table_hbm, out_hbm, idx_v, rows_v, g0, g1, s0, s1):
    c = lax.axis_index("c")
    s = lax.axis_index("s")
    wid = s * _NC + c
    gsem = (g0, g1)
    ssem = (s0, s1)
    pltpu.sync_copy(idx_hbm.at[wid], idx_v)
    base = wid * PER_W

    def fire_group(gidx, b):
        for k in range(K):
            pltpu.async_copy(
                table_hbm.at[idx_v.at[gidx * K + k]],
                rows_v.at[b, pl.ds(k * CHUNK, CHUNK)],
                gsem[b],
            )

    def drain_gathers(b):
        pltpu.make_async_copy(
            table_hbm.at[pl.ds(0, GROUP)], rows_v.at[b], gsem[b]
        ).wait()

    def drain_store(b):
        pltpu.make_async_copy(
            rows_v.at[b], out_hbm.at[pl.ds(0, GROUP)], ssem[b]
        ).wait()

    for b in range(NB):
        fire_group(b, b)

    def body(i, carry):
        for b in range(NB):
            g = i * NB + b
            drain_gathers(b)
            pltpu.async_copy(
                rows_v.at[b],
                out_hbm.at[pl.ds(base + g * GROUP, GROUP)],
                ssem[b],
            )
            gn = g + NB

            @pl.when(gn < NGROUP)
            def _refill():
                drain_store(b)
                fire_group(gn, b)

        return carry

    lax.fori_loop(0, NITER, body, 0)
    for b in range(NB):
        drain_store(b)


FBT = 16000
FGRID = TREE_NODE_NUM // FBT


def _fmt_body(t_ref, out_ref):
    x = t_ref[...]
    xt = jnp.swapaxes(x, 0, 1)
    xt3 = xt.reshape(FBT // 2, 2, EMBED_DIM)
    out_ref[...] = jnp.concatenate([xt3[:, 0, :], xt3[:, 1, :]], axis=1)


_tc_format = pl.pallas_call(
    _fmt_body,
    grid=(FGRID,),
    in_specs=[pl.BlockSpec((EMBED_DIM, FBT), lambda i: (0, i))],
    out_specs=pl.BlockSpec((FBT // 2, 2 * EMBED_DIM), lambda i: (i, 0)),
    out_shape=jax.ShapeDtypeStruct(
        (TREE_NODE_NUM // 2, 2 * EMBED_DIM), jnp.float32
    ),
)


OBT = 128
OGRID = BATCH // OBT


def _out_body(in_ref, out_ref):
    x = in_ref[...]
    x3 = x.reshape(OBT, HIST // 2, 128)
    for hh in range(HIST // 2):
        t = jnp.swapaxes(x3[:, hh, :], 0, 1)
        out_ref[2 * hh, :, :] = t[0:EMBED_DIM]
        out_ref[2 * hh + 1, :, :] = t[EMBED_DIM:]


_tc_out_format = pl.pallas_call(
    _out_body,
    grid=(OGRID,),
    in_specs=[pl.BlockSpec((OBT * HIST // 2, 128), lambda i: (i, 0))],
    out_specs=pl.BlockSpec((HIST, EMBED_DIM, OBT), lambda i: (0, 0, i)),
    out_shape=jax.ShapeDtypeStruct((HIST, EMBED_DIM, BATCH), jnp.float32),
)


def kernel(itemIDs, emb_table):
    idx = itemIDs.reshape(_NW, NCHUNK, CHUNK).astype(jnp.int32)
    pairs = _tc_format(emb_table.T)
    table_rm = pairs.reshape(TREE_NODE_NUM, EMBED_DIM)
    flat = _sc_gather(idx, table_rm)
    out3 = _tc_out_format(flat.reshape(TOTAL // 2, 2 * EMBED_DIM))
    return jnp.transpose(out3, (2, 0, 1))


# speedup vs baseline: 1.4488x; 1.0457x over previous
import functools

import jax
import jax.numpy as jnp
from jax import lax
from jax.experimental import pallas as pl
from jax.experimental.pallas import tpu as pltpu
from jax.experimental.pallas import tpu_sc as plsc

TREE_NODE_NUM = 2000000
EMBED_DIM = 64
BATCH = 16384
HIST = 50
TOTAL = BATCH * HIST

_info = plsc.get_sparse_core_info()
_NC = _info.num_cores
_NS = _info.num_subcores
_NW = _NC * _NS

CHUNK = 128
K = 4
GROUP = K * CHUNK
NB = 2
PER_W = TOTAL // _NW
NCHUNK = PER_W // CHUNK
NGROUP = PER_W // GROUP
NITER = NGROUP // NB

_mesh = plsc.VectorSubcoreMesh(core_axis_name="c", subcore_axis_name="s")


@functools.partial(
    pl.kernel,
    mesh=_mesh,
    out_type=jax.ShapeDtypeStruct((TOTAL, EMBED_DIM), jnp.float32),
    scratch_types=[
        pltpu.VMEM((NCHUNK, CHUNK), jnp.int32),
        pltpu.VMEM((NB, GROUP, EMBED_DIM), jnp.float32),
        pltpu.SemaphoreType.DMA,
        pltpu.SemaphoreType.DMA,
        pltpu.SemaphoreType.DMA,
        pltpu.SemaphoreType.DMA,
    ],
    compiler_params=pltpu.CompilerParams(use_tc_tiling_on_sc=False),
)
def _sc_gather(idx_hbm, table_hbm, out_hbm, idx_v, rows_v, g0, g1, s0, s1):
    c = lax.axis_index("c")
    s = lax.axis_index("s")
    wid = s * _NC + c
    gsem = (g0, g1)
    ssem = (s0, s1)
    pltpu.sync_copy(idx_hbm.at[wid], idx_v)
    base = wid * PER_W

    def fire_group(gidx, b):
        for k in range(K):
            pltpu.async_copy(
                table_hbm.at[idx_v.at[gidx * K + k]],
                rows_v.at[b, pl.ds(k * CHUNK, CHUNK)],
                gsem[b],
            )

    def drain_gathers(b):
        pltpu.make_async_copy(
            table_hbm.at[pl.ds(0, GROUP)], rows_v.at[b], gsem[b]
        ).wait()

    def drain_store(b):
        pltpu.make_async_copy(
            rows_v.at[b], out_hbm.at[pl.ds(0, GROUP)], ssem[b]
        ).wait()

    for b in range(NB):
        fire_group(b, b)

    def body(i, carry):
        for b in range(NB):
            g = i * NB + b
            drain_gathers(b)
            pltpu.async_copy(
                rows_v.at[b],
                out_hbm.at[pl.ds(base + g * GROUP, GROUP)],
                ssem[b],
            )
            gn = g + NB

            @pl.when(gn < NGROUP)
            def _refill():
                drain_store(b)
                fire_group(gn, b)

        return carry

    lax.fori_loop(0, NITER, body, 0)
    for b in range(NB):
        drain_store(b)


FBT = 16000
FGRID = TREE_NODE_NUM // FBT


def _fmt_body(t_ref, out_ref):
    x = t_ref[...]
    xt = jnp.swapaxes(x, 0, 1)
    xt3 = xt.reshape(FBT // 2, 2, EMBED_DIM)
    out_ref[...] = jnp.concatenate([xt3[:, 0, :], xt3[:, 1, :]], axis=1)


_tc_format = pl.pallas_call(
    _fmt_body,
    grid=(FGRID,),
    in_specs=[pl.BlockSpec((EMBED_DIM, FBT), lambda i: (0, i))],
    out_specs=pl.BlockSpec((FBT // 2, 2 * EMBED_DIM), lambda i: (i, 0)),
    out_shape=jax.ShapeDtypeStruct(
        (TREE_NODE_NUM // 2, 2 * EMBED_DIM), jnp.float32
    ),
)


OBT = 512
OGRID = BATCH // OBT


def _out_body(in_ref, out_ref):
    x = in_ref[...]
    x3 = x.reshape(OBT, HIST // 2, 128)
    for hh in range(HIST // 2):
        t = jnp.swapaxes(x3[:, hh, :], 0, 1)
        out_ref[2 * hh, :, :] = t[0:EMBED_DIM]
        out_ref[2 * hh + 1, :, :] = t[EMBED_DIM:]


_tc_out_format = pl.pallas_call(
    _out_body,
    grid=(OGRID,),
    in_specs=[pl.BlockSpec((OBT * HIST // 2, 128), lambda i: (i, 0))],
    out_specs=pl.BlockSpec((HIST, EMBED_DIM, OBT), lambda i: (0, 0, i)),
    out_shape=jax.ShapeDtypeStruct((HIST, EMBED_DIM, BATCH), jnp.float32),
)


def kernel(itemIDs, emb_table):
    idx = itemIDs.reshape(_NW, NCHUNK, CHUNK).astype(jnp.int32)
    pairs = _tc_format(emb_table.T)
    table_rm = pairs.reshape(TREE_NODE_NUM, EMBED_DIM)
    flat = _sc_gather(idx, table_rm)
    out3 = _tc_out_format(flat.reshape(TOTAL // 2, 2 * EMBED_DIM))
    return jnp.transpose(out3, (2, 0, 1))
